# baseline (speedup 1.0000x reference)
import functools

import jax
import jax.numpy as jnp
from jax import lax
from jax.experimental import pallas as pl
from jax.experimental.pallas import tpu as pltpu
from jax.experimental.pallas import tpu_sc as plsc

NC = 2
NS = 16
NW = NC * NS
LANE = 16
W = 16
WP = 8
CHUNK = 128


def _sc_mesh():
    return plsc.VectorSubcoreMesh(
        core_axis_name="c", subcore_axis_name="s", num_cores=NC, num_subcores=NS
    )


NB = 8


def _adjust_idx(row_v, col_v, idx_buf, j, trash):
    for k in range(CHUNK // LANE):
        r = row_v[j, pl.ds(k * LANE, LANE)]
        c = col_v[j, pl.ds(k * LANE, LANE)]
        idx_buf[pl.ds(k * LANE, LANE)] = jnp.where(r == c, jnp.int32(trash), c)


def _make_deg_kernel(n_chunks, npad, rpt, trash):
    ept = n_chunks * CHUNK

    scratch = [
        pltpu.VMEM((ept,), jnp.int32),
        pltpu.VMEM((ept,), jnp.int32),
        pltpu.VMEM((npad,), jnp.float32),
        pltpu.VMEM((NS * rpt,), jnp.float32),
        pltpu.VMEM((rpt,), jnp.float32),
        pltpu.VMEM_SHARED((NS, npad), jnp.float32),
    ]

    @functools.partial(
        pl.kernel,
        mesh=_sc_mesh(),
        out_type=jax.ShapeDtypeStruct((NC, NS, rpt), jnp.float32),
        scratch_types=scratch,
        compiler_params=pltpu.CompilerParams(use_tc_tiling_on_sc=False,
                                             needs_layout_passes=False),
    )
    def deg_kernel(row_hbm, col_hbm, out_hbm,
                   row_v, col_v, hist, peers, red, stage):
        cid = lax.axis_index("c")
        sid = lax.axis_index("s")
        wid = cid * NS + sid

        pltpu.sync_copy(row_hbm.at[pl.ds(wid * ept, ept)], row_v)
        pltpu.sync_copy(col_hbm.at[pl.ds(wid * ept, ept)], col_v)

        zero16 = jnp.zeros((LANE,), jnp.float32)

        def zloop(i, _):
            hist[pl.ds(i * LANE, LANE)] = zero16
            return 0

        lax.fori_loop(0, npad // LANE, zloop, 0)

        one16 = jnp.full((LANE,), 1.0, jnp.float32)

        def count(q, _):
            r = row_v[pl.ds(q * LANE, LANE)]
            c = col_v[pl.ds(q * LANE, LANE)]
            adj = jnp.where(r == c, jnp.int32(trash), c)
            plsc.addupdate_scatter(hist, [adj], one16)
            return 0

        lax.fori_loop(0, ept // LANE, count, 0)

        pltpu.sync_copy(hist, stage.at[sid])
        plsc.subcore_barrier()
        for s in range(NS):
            pltpu.sync_copy(stage.at[s, pl.ds(sid * rpt, rpt)],
                            peers.at[pl.ds(s * rpt, rpt)])

        def radd(i, _):
            acc = peers[pl.ds(i * LANE, LANE)]
            for s in range(1, NS):
                acc = acc + peers[pl.ds(s * rpt + i * LANE, LANE)]
            red[pl.ds(i * LANE, LANE)] = acc
            return 0

        lax.fori_loop(0, rpt // LANE, radd, 0)

        pltpu.sync_copy(red, out_hbm.at[cid, sid])

    return deg_kernel


def _make_prop_kernel(n_chunks, npad, rpt, trash):
    groups = n_chunks // NB

    scratch = [
        pltpu.VMEM((n_chunks, CHUNK), jnp.int32),
        pltpu.VMEM((n_chunks, CHUNK), jnp.int32),
    ]
    scratch += [pltpu.VMEM((CHUNK,), jnp.int32) for _ in range(NB)]
    scratch += [pltpu.VMEM((CHUNK, WP), jnp.float32) for _ in range(NB)]
    scratch += [pltpu.VMEM_SHARED((npad, WP), jnp.float32)]
    scratch += [pltpu.VMEM_SHARED((npad, WP), jnp.float32)]
    scratch += [pltpu.SemaphoreType.DMA for _ in range(NB)]
    scratch += [pltpu.SemaphoreType.DMA for _ in range(NB)]

    @functools.partial(
        pl.kernel,
        mesh=_sc_mesh(),
        out_type=jax.ShapeDtypeStruct((NC, npad, WP), jnp.float32),
        scratch_types=scratch,
        compiler_params=pltpu.CompilerParams(use_tc_tiling_on_sc=False),
    )
    def prop_kernel(row_hbm, col_hbm, zeros_hbm, u_hbm, out_hbm, *refs):
        row_v, col_v = refs[0], refs[1]
        sidx = refs[2:2 + NB]
        gat = refs[2 + NB:2 + 2 * NB]
        acc = refs[2 + 2 * NB]
        u_sp = refs[3 + 2 * NB]
        gsem = refs[4 + 2 * NB:4 + 3 * NB]
        ssem = refs[4 + 3 * NB:4 + 4 * NB]

        cid = lax.axis_index("c")
        sid = lax.axis_index("s")
        wid = cid * NS + sid

        pltpu.sync_copy(zeros_hbm.at[pl.ds(sid * rpt, rpt)],
                        acc.at[pl.ds(sid * rpt, rpt)])
        pltpu.sync_copy(u_hbm.at[pl.ds(sid * rpt, rpt)],
                        u_sp.at[pl.ds(sid * rpt, rpt)])

        pltpu.sync_copy(row_hbm.at[pl.ds(wid * n_chunks, n_chunks)], row_v)
        pltpu.sync_copy(col_hbm.at[pl.ds(wid * n_chunks, n_chunks)], col_v)

        plsc.subcore_barrier()

        for b in range(NB):
            _adjust_idx(row_v, col_v, sidx[b], b, trash)
            pltpu.async_copy(u_sp.at[row_v.at[b]], gat[b], gsem[b])

        def group(g, _):
            for b in range(NB):
                j = g * NB + b
                jn = j + NB
                pltpu.make_async_copy(u_sp.at[row_v.at[j]], gat[b],
                                      gsem[b]).wait()
                pltpu.async_copy(gat[b], acc.at[sidx[b]], ssem[b], add=True)
                pltpu.make_async_copy(gat[b], acc.at[sidx[b]], ssem[b]).wait()
                _adjust_idx(row_v, col_v, sidx[b], jn, trash)
                pltpu.async_copy(u_sp.at[row_v.at[jn]], gat[b], gsem[b])
            return 0

        lax.fori_loop(0, groups - 1, group, 0)

        for b in range(NB):
            j = (groups - 1) * NB + b
            pltpu.make_async_copy(u_sp.at[row_v.at[j]], gat[b], gsem[b]).wait()
            pltpu.sync_copy(gat[b], acc.at[sidx[b]], add=True)

        plsc.subcore_barrier()
        pltpu.sync_copy(acc.at[pl.ds(sid * rpt, rpt)],
                        out_hbm.at[cid, pl.ds(sid * rpt, rpt)])

    return prop_kernel


def _mm_body(x_ref, w_ref, o_ref):
    o_ref[...] = jnp.dot(x_ref[...], w_ref[...],
                         preferred_element_type=jnp.float32)


def _scale_body(degp_ref, z_ref, u_ref):
    n_rows = z_ref.shape[0]
    npad_rows = u_ref.shape[0]
    cnt = degp_ref[0, :n_rows] + degp_ref[1, :n_rows]
    deg = cnt + 1.0
    u_ref[0:n_rows] = lax.rsqrt(deg) * z_ref[...]
    u_ref[n_rows:npad_rows] = jnp.zeros(
        (npad_rows - n_rows, u_ref.shape[1]), jnp.float32)


def _final_body(degp_ref, tp_ref, z_ref, a_ref, b_ref, o_ref):
    n_rows = z_ref.shape[0]
    cnt = degp_ref[0, :n_rows] + degp_ref[1, :n_rows]
    deg = cnt + 1.0
    t = tp_ref[0, :n_rows] + tp_ref[1, :n_rows]
    agg = lax.rsqrt(deg) * t + z_ref[...] / deg
    o_ref[...] = jnp.dot(agg, a_ref[...],
                         preferred_element_type=jnp.float32) + b_ref[...]


def kernel(x, edge_index, B_w, A_w, bias):
    n, d_in = x.shape
    d_out = A_w.shape[0]
    r = B_w.shape[0]
    e = edge_index.shape[1]

    npad = ((n + 1 + NS * LANE - 1) // (NS * LANE)) * (NS * LANE)
    rpt = npad // NS
    n_chunks = (e + NW * CHUNK - 1) // (NW * CHUNK)
    n_chunks = ((n_chunks + 7) // 8) * 8
    epad = NW * CHUNK * n_chunks
    trash = n

    row = jnp.concatenate(
        [edge_index[0], jnp.zeros((epad - e,), dtype=jnp.int32)])
    col = jnp.concatenate(
        [edge_index[1], jnp.zeros((epad - e,), dtype=jnp.int32)])
    row2 = row.reshape(NW * n_chunks, CHUNK)
    col2 = col.reshape(NW * n_chunks, CHUNK)

    bw_pad = jnp.zeros((d_in, WP), jnp.float32).at[:, :r].set(B_w.T)
    a_pad = jnp.zeros((WP, d_out), jnp.float32).at[:r, :].set(A_w.T)
    zeros8 = jnp.zeros((npad, WP), jnp.float32)

    z = pl.pallas_call(
        _mm_body,
        out_shape=jax.ShapeDtypeStruct((n, WP), jnp.float32),
    )(x, bw_pad)

    degp = _make_deg_kernel(n_chunks, npad, rpt, trash)(row, col)
    degp = degp.reshape(NC, npad, 1)

    u = pl.pallas_call(
        _scale_body,
        out_shape=jax.ShapeDtypeStruct((npad, WP), jnp.float32),
    )(degp, z)

    tp = _make_prop_kernel(n_chunks, npad, rpt, trash)(row2, col2, zeros8, u)

    return pl.pallas_call(
        _final_body,
        out_shape=jax.ShapeDtypeStruct((n, d_out), jnp.float32),
    )(degp, tp, z, a_pad, bias.reshape(1, d_out))


# speedup vs baseline: 1.0269x; 1.0269x over previous
import functools

import jax
import jax.numpy as jnp
from jax import lax
from jax.experimental import pallas as pl
from jax.experimental.pallas import tpu as pltpu
from jax.experimental.pallas import tpu_sc as plsc

NC = 2
NS = 16
NW = NC * NS
LANE = 16
W = 16
WP = 8
CHUNK = 128


def _sc_mesh():
    return plsc.VectorSubcoreMesh(
        core_axis_name="c", subcore_axis_name="s", num_cores=NC, num_subcores=NS
    )


NB = 8


def _adjust_idx(row_v, col_v, idx_buf, j, trash):
    for k in range(CHUNK // LANE):
        r = row_v[j, pl.ds(k * LANE, LANE)]
        c = col_v[j, pl.ds(k * LANE, LANE)]
        idx_buf[pl.ds(k * LANE, LANE)] = jnp.where(r == c, jnp.int32(trash), c)


def _make_deg_kernel(n_chunks, npad, rpt, trash):
    groups = n_chunks // NB

    scratch = [
        pltpu.VMEM((n_chunks, CHUNK), jnp.int32),
        pltpu.VMEM((n_chunks, CHUNK), jnp.int32),
    ]
    scratch += [pltpu.VMEM((CHUNK,), jnp.int32) for _ in range(NB)]
    scratch += [
        pltpu.VMEM((CHUNK, W), jnp.float32),
        pltpu.VMEM_SHARED((npad, W), jnp.float32),
    ]
    scratch += [pltpu.SemaphoreType.DMA for _ in range(NB)]

    @functools.partial(
        pl.kernel,
        mesh=_sc_mesh(),
        out_type=jax.ShapeDtypeStruct((NC, npad, W), jnp.float32),
        scratch_types=scratch,
        compiler_params=pltpu.CompilerParams(use_tc_tiling_on_sc=False),
    )
    def deg_kernel(row_hbm, col_hbm, zeros_hbm, out_hbm, *refs):
        row_v, col_v = refs[0], refs[1]
        sidx = refs[2:2 + NB]
        val_v = refs[2 + NB]
        acc = refs[3 + NB]
        sems = refs[4 + NB:4 + 2 * NB]

        cid = lax.axis_index("c")
        sid = lax.axis_index("s")
        wid = cid * NS + sid

        pltpu.sync_copy(zeros_hbm.at[pl.ds(sid * rpt, rpt)],
                        acc.at[pl.ds(sid * rpt, rpt)])

        pltpu.sync_copy(row_hbm.at[pl.ds(wid * n_chunks, n_chunks)], row_v)
        pltpu.sync_copy(col_hbm.at[pl.ds(wid * n_chunks, n_chunks)], col_v)

        one0 = jnp.where(lax.iota(jnp.int32, LANE) == 0,
                         jnp.float32(1.0), jnp.float32(0.0))

        def fill(i, _):
            val_v[i, :] = one0
            return 0

        lax.fori_loop(0, CHUNK, fill, 0)

        plsc.subcore_barrier()

        for b in range(NB):
            _adjust_idx(row_v, col_v, sidx[b], b, trash)
            pltpu.async_copy(val_v, acc.at[sidx[b]], sems[b], add=True)

        def group(g, _):
            for b in range(NB):
                pltpu.make_async_copy(val_v, acc.at[sidx[b]], sems[b]).wait()
                _adjust_idx(row_v, col_v, sidx[b], (g + 1) * NB + b, trash)
                pltpu.async_copy(val_v, acc.at[sidx[b]], sems[b], add=True)
            return 0

        lax.fori_loop(0, groups - 1, group, 0)

        for b in range(NB):
            pltpu.make_async_copy(val_v, acc.at[sidx[b]], sems[b]).wait()

        plsc.subcore_barrier()
        pltpu.sync_copy(acc.at[pl.ds(sid * rpt, rpt)],
                        out_hbm.at[cid, pl.ds(sid * rpt, rpt)])

    return deg_kernel


def _make_prop_kernel(n_chunks, npad, rpt, trash):
    groups = n_chunks // NB

    scratch = [
        pltpu.VMEM((n_chunks, CHUNK), jnp.int32),
        pltpu.VMEM((n_chunks, CHUNK), jnp.int32),
    ]
    scratch += [pltpu.VMEM((CHUNK,), jnp.int32) for _ in range(NB)]
    scratch += [pltpu.VMEM((CHUNK, WP), jnp.float32) for _ in range(NB)]
    scratch += [pltpu.VMEM_SHARED((npad, WP), jnp.float32)]
    scratch += [pltpu.VMEM_SHARED((npad, WP), jnp.float32)]
    scratch += [pltpu.SemaphoreType.DMA for _ in range(NB)]
    scratch += [pltpu.SemaphoreType.DMA for _ in range(NB)]

    @functools.partial(
        pl.kernel,
        mesh=_sc_mesh(),
        out_type=jax.ShapeDtypeStruct((NC, npad, WP), jnp.float32),
        scratch_types=scratch,
        compiler_params=pltpu.CompilerParams(use_tc_tiling_on_sc=False),
    )
    def prop_kernel(row_hbm, col_hbm, zeros_hbm, u_hbm, out_hbm, *refs):
        row_v, col_v = refs[0], refs[1]
        sidx = refs[2:2 + NB]
        gat = refs[2 + NB:2 + 2 * NB]
        acc = refs[2 + 2 * NB]
        u_sp = refs[3 + 2 * NB]
        gsem = refs[4 + 2 * NB:4 + 3 * NB]
        ssem = refs[4 + 3 * NB:4 + 4 * NB]

        cid = lax.axis_index("c")
        sid = lax.axis_index("s")
        wid = cid * NS + sid

        pltpu.sync_copy(zeros_hbm.at[pl.ds(sid * rpt, rpt)],
                        acc.at[pl.ds(sid * rpt, rpt)])
        pltpu.sync_copy(u_hbm.at[pl.ds(sid * rpt, rpt)],
                        u_sp.at[pl.ds(sid * rpt, rpt)])

        pltpu.sync_copy(row_hbm.at[pl.ds(wid * n_chunks, n_chunks)], row_v)
        pltpu.sync_copy(col_hbm.at[pl.ds(wid * n_chunks, n_chunks)], col_v)

        plsc.subcore_barrier()

        for b in range(NB):
            _adjust_idx(row_v, col_v, sidx[b], b, trash)
            pltpu.async_copy(u_sp.at[row_v.at[b]], gat[b], gsem[b])

        def group(g, _):
            for b in range(NB):
                j = g * NB + b
                jn = j + NB
                pltpu.make_async_copy(u_sp.at[row_v.at[j]], gat[b],
                                      gsem[b]).wait()
                pltpu.async_copy(gat[b], acc.at[sidx[b]], ssem[b], add=True)
                pltpu.make_async_copy(gat[b], acc.at[sidx[b]], ssem[b]).wait()
                _adjust_idx(row_v, col_v, sidx[b], jn, trash)
                pltpu.async_copy(u_sp.at[row_v.at[jn]], gat[b], gsem[b])
            return 0

        lax.fori_loop(0, groups - 1, group, 0)

        for b in range(NB):
            j = (groups - 1) * NB + b
            pltpu.make_async_copy(u_sp.at[row_v.at[j]], gat[b], gsem[b]).wait()
            pltpu.sync_copy(gat[b], acc.at[sidx[b]], add=True)

        plsc.subcore_barrier()
        pltpu.sync_copy(acc.at[pl.ds(sid * rpt, rpt)],
                        out_hbm.at[cid, pl.ds(sid * rpt, rpt)])

    return prop_kernel


def _mm_body(x_ref, w_ref, o_ref):
    o_ref[...] = jnp.dot(x_ref[...], w_ref[...],
                         preferred_element_type=jnp.float32)


def _scale_body(degp_ref, z_ref, u_ref):
    n_rows = z_ref.shape[0]
    npad_rows = u_ref.shape[0]
    cnt = degp_ref[0, :n_rows] + degp_ref[1, :n_rows]
    deg = cnt[:, 0:1] + 1.0
    u_ref[0:n_rows] = lax.rsqrt(deg) * z_ref[...]
    u_ref[n_rows:npad_rows] = jnp.zeros(
        (npad_rows - n_rows, u_ref.shape[1]), jnp.float32)


def _final_body(degp_ref, tp_ref, z_ref, a_ref, b_ref, o_ref):
    n_rows = z_ref.shape[0]
    cnt = degp_ref[0, :n_rows] + degp_ref[1, :n_rows]
    deg = cnt[:, 0:1] + 1.0
    t = tp_ref[0, :n_rows] + tp_ref[1, :n_rows]
    agg = lax.rsqrt(deg) * t + z_ref[...] / deg
    o_ref[...] = jnp.dot(agg, a_ref[...],
                         preferred_element_type=jnp.float32) + b_ref[...]


def kernel(x, edge_index, B_w, A_w, bias):
    n, d_in = x.shape
    d_out = A_w.shape[0]
    r = B_w.shape[0]
    e = edge_index.shape[1]

    npad = ((n + 1 + NS * 8 - 1) // (NS * 8)) * (NS * 8)
    rpt = npad // NS
    n_chunks = (e + NW * CHUNK - 1) // (NW * CHUNK)
    n_chunks = ((n_chunks + 7) // 8) * 8
    epad = NW * CHUNK * n_chunks
    trash = n

    row = jnp.concatenate(
        [edge_index[0], jnp.zeros((epad - e,), dtype=jnp.int32)])
    col = jnp.concatenate(
        [edge_index[1], jnp.zeros((epad - e,), dtype=jnp.int32)])
    row2 = row.reshape(NW * n_chunks, CHUNK)
    col2 = col.reshape(NW * n_chunks, CHUNK)

    bw_pad = jnp.zeros((d_in, WP), jnp.float32).at[:, :r].set(B_w.T)
    a_pad = jnp.zeros((WP, d_out), jnp.float32).at[:r, :].set(A_w.T)
    zeros16 = jnp.zeros((npad, W), jnp.float32)
    zeros8 = jnp.zeros((npad, WP), jnp.float32)

    z = pl.pallas_call(
        _mm_body,
        out_shape=jax.ShapeDtypeStruct((n, WP), jnp.float32),
    )(x, bw_pad)

    degp = _make_deg_kernel(n_chunks, npad, rpt, trash)(row2, col2, zeros16)

    u = pl.pallas_call(
        _scale_body,
        out_shape=jax.ShapeDtypeStruct((npad, WP), jnp.float32),
    )(degp, z)

    tp = _make_prop_kernel(n_chunks, npad, rpt, trash)(row2, col2, zeros8, u)

    return pl.pallas_call(
        _final_body,
        out_shape=jax.ShapeDtypeStruct((n, d_out), jnp.float32),
    )(degp, tp, z, a_pad, bias.reshape(1, d_out))


# speedup vs baseline: 1.0666x; 1.0387x over previous
import functools

import jax
import jax.numpy as jnp
from jax import lax
from jax.experimental import pallas as pl
from jax.experimental.pallas import tpu as pltpu
from jax.experimental.pallas import tpu_sc as plsc

NC = 2
NS = 16
NW = NC * NS
LANE = 16
W = 16
WP = 8
CHUNK = 128


def _sc_mesh():
    return plsc.VectorSubcoreMesh(
        core_axis_name="c", subcore_axis_name="s", num_cores=NC, num_subcores=NS
    )


NB = 8


def _adjust_idx(row_v, col_v, idx_buf, j, trash):
    for k in range(CHUNK // LANE):
        r = row_v[j, pl.ds(k * LANE, LANE)]
        c = col_v[j, pl.ds(k * LANE, LANE)]
        idx_buf[pl.ds(k * LANE, LANE)] = jnp.where(r == c, jnp.int32(trash), c)


def _make_deg_kernel(n_chunks, npad, rpt, trash):
    groups = n_chunks // NB

    scratch = [
        pltpu.VMEM((n_chunks, CHUNK), jnp.int32),
        pltpu.VMEM((n_chunks, CHUNK), jnp.int32),
    ]
    scratch += [pltpu.VMEM((CHUNK,), jnp.int32) for _ in range(NB)]
    scratch += [
        pltpu.VMEM((CHUNK, W), jnp.float32),
        pltpu.VMEM_SHARED((npad, W), jnp.float32),
    ]
    scratch += [pltpu.SemaphoreType.DMA for _ in range(NB)]

    @functools.partial(
        pl.kernel,
        mesh=_sc_mesh(),
        out_type=jax.ShapeDtypeStruct((NC, npad, W), jnp.float32),
        scratch_types=scratch,
        compiler_params=pltpu.CompilerParams(use_tc_tiling_on_sc=False),
    )
    def deg_kernel(row_hbm, col_hbm, zeros_hbm, out_hbm, *refs):
        row_v, col_v = refs[0], refs[1]
        sidx = refs[2:2 + NB]
        val_v = refs[2 + NB]
        acc = refs[3 + NB]
        sems = refs[4 + NB:4 + 2 * NB]

        cid = lax.axis_index("c")
        sid = lax.axis_index("s")
        wid = cid * NS + sid

        c1 = pltpu.async_copy(zeros_hbm.at[pl.ds(sid * rpt, rpt)],
                              acc.at[pl.ds(sid * rpt, rpt)], sems[0])
        c2 = pltpu.async_copy(row_hbm.at[pl.ds(wid * n_chunks, n_chunks)],
                              row_v, sems[1])
        c3 = pltpu.async_copy(col_hbm.at[pl.ds(wid * n_chunks, n_chunks)],
                              col_v, sems[2])

        one0 = jnp.where(lax.iota(jnp.int32, LANE) == 0,
                         jnp.float32(1.0), jnp.float32(0.0))

        def fill(i, _):
            val_v[i, :] = one0
            return 0

        lax.fori_loop(0, CHUNK, fill, 0)

        c1.wait()
        c2.wait()
        c3.wait()

        plsc.subcore_barrier()

        for b in range(NB):
            _adjust_idx(row_v, col_v, sidx[b], b, trash)
            pltpu.async_copy(val_v, acc.at[sidx[b]], sems[b], add=True)

        def group(g, _):
            for b in range(NB):
                pltpu.make_async_copy(val_v, acc.at[sidx[b]], sems[b]).wait()
                _adjust_idx(row_v, col_v, sidx[b], (g + 1) * NB + b, trash)
                pltpu.async_copy(val_v, acc.at[sidx[b]], sems[b], add=True)
            return 0

        lax.fori_loop(0, groups - 1, group, 0)

        for b in range(NB):
            pltpu.make_async_copy(val_v, acc.at[sidx[b]], sems[b]).wait()

        plsc.subcore_barrier()
        pltpu.sync_copy(acc.at[pl.ds(sid * rpt, rpt)],
                        out_hbm.at[cid, pl.ds(sid * rpt, rpt)])

    return deg_kernel


def _make_prop_kernel(n_chunks, npad, rpt, trash):
    groups = n_chunks // NB

    scratch = [
        pltpu.VMEM((n_chunks, CHUNK), jnp.int32),
        pltpu.VMEM((n_chunks, CHUNK), jnp.int32),
    ]
    scratch += [pltpu.VMEM((CHUNK,), jnp.int32) for _ in range(NB)]
    scratch += [pltpu.VMEM((CHUNK, WP), jnp.float32) for _ in range(NB)]
    scratch += [pltpu.VMEM_SHARED((npad, WP), jnp.float32)]
    scratch += [pltpu.VMEM_SHARED((npad, WP), jnp.float32)]
    scratch += [pltpu.SemaphoreType.DMA for _ in range(NB)]
    scratch += [pltpu.SemaphoreType.DMA for _ in range(NB)]

    @functools.partial(
        pl.kernel,
        mesh=_sc_mesh(),
        out_type=jax.ShapeDtypeStruct((NC, npad, WP), jnp.float32),
        scratch_types=scratch,
        compiler_params=pltpu.CompilerParams(use_tc_tiling_on_sc=False),
    )
    def prop_kernel(row_hbm, col_hbm, zeros_hbm, u_hbm, out_hbm, *refs):
        row_v, col_v = refs[0], refs[1]
        sidx = refs[2:2 + NB]
        gat = refs[2 + NB:2 + 2 * NB]
        acc = refs[2 + 2 * NB]
        u_sp = refs[3 + 2 * NB]
        gsem = refs[4 + 2 * NB:4 + 3 * NB]
        ssem = refs[4 + 3 * NB:4 + 4 * NB]

        cid = lax.axis_index("c")
        sid = lax.axis_index("s")
        wid = cid * NS + sid

        c1 = pltpu.async_copy(zeros_hbm.at[pl.ds(sid * rpt, rpt)],
                              acc.at[pl.ds(sid * rpt, rpt)], gsem[0])
        c2 = pltpu.async_copy(u_hbm.at[pl.ds(sid * rpt, rpt)],
                              u_sp.at[pl.ds(sid * rpt, rpt)], gsem[1])
        c3 = pltpu.async_copy(row_hbm.at[pl.ds(wid * n_chunks, n_chunks)],
                              row_v, gsem[2])
        c4 = pltpu.async_copy(col_hbm.at[pl.ds(wid * n_chunks, n_chunks)],
                              col_v, gsem[3])
        c1.wait()
        c2.wait()
        c3.wait()
        c4.wait()

        plsc.subcore_barrier()

        for b in range(NB):
            _adjust_idx(row_v, col_v, sidx[b], b, trash)
            pltpu.async_copy(u_sp.at[row_v.at[b]], gat[b], gsem[b])

        def group(g, _):
            for b in range(NB):
                j = g * NB + b
                jn = j + NB
                pltpu.make_async_copy(u_sp.at[row_v.at[j]], gat[b],
                                      gsem[b]).wait()
                pltpu.async_copy(gat[b], acc.at[sidx[b]], ssem[b], add=True)
                pltpu.make_async_copy(gat[b], acc.at[sidx[b]], ssem[b]).wait()
                _adjust_idx(row_v, col_v, sidx[b], jn, trash)
                pltpu.async_copy(u_sp.at[row_v.at[jn]], gat[b], gsem[b])
            return 0

        lax.fori_loop(0, groups - 1, group, 0)

        for b in range(NB):
            j = (groups - 1) * NB + b
            pltpu.make_async_copy(u_sp.at[row_v.at[j]], gat[b], gsem[b]).wait()
            pltpu.sync_copy(gat[b], acc.at[sidx[b]], add=True)

        plsc.subcore_barrier()
        pltpu.sync_copy(acc.at[pl.ds(sid * rpt, rpt)],
                        out_hbm.at[cid, pl.ds(sid * rpt, rpt)])

    return prop_kernel


def _mm_body(x_ref, w_ref, o_ref):
    o_ref[...] = jnp.dot(x_ref[...], w_ref[...],
                         preferred_element_type=jnp.float32)


def _scale_body(degp_ref, z_ref, u_ref):
    n_rows = z_ref.shape[0]
    npad_rows = u_ref.shape[0]
    cnt = degp_ref[0, :n_rows] + degp_ref[1, :n_rows]
    deg = cnt[:, 0:1] + 1.0
    u_ref[0:n_rows] = lax.rsqrt(deg) * z_ref[...]
    u_ref[n_rows:npad_rows] = jnp.zeros(
        (npad_rows - n_rows, u_ref.shape[1]), jnp.float32)


def _final_body(degp_ref, tp_ref, z_ref, a_ref, b_ref, o_ref):
    n_rows = z_ref.shape[0]
    cnt = degp_ref[0, :n_rows] + degp_ref[1, :n_rows]
    deg = cnt[:, 0:1] + 1.0
    t = tp_ref[0, :n_rows] + tp_ref[1, :n_rows]
    agg = lax.rsqrt(deg) * t + z_ref[...] / deg
    o_ref[...] = jnp.dot(agg, a_ref[...],
                         preferred_element_type=jnp.float32) + b_ref[...]


def kernel(x, edge_index, B_w, A_w, bias):
    n, d_in = x.shape
    d_out = A_w.shape[0]
    r = B_w.shape[0]
    e = edge_index.shape[1]

    npad = ((n + 1 + NS * 8 - 1) // (NS * 8)) * (NS * 8)
    rpt = npad // NS
    n_chunks = (e + NW * CHUNK - 1) // (NW * CHUNK)
    n_chunks = ((n_chunks + 7) // 8) * 8
    epad = NW * CHUNK * n_chunks
    trash = n

    row = jnp.concatenate(
        [edge_index[0], jnp.zeros((epad - e,), dtype=jnp.int32)])
    col = jnp.concatenate(
        [edge_index[1], jnp.zeros((epad - e,), dtype=jnp.int32)])
    row2 = row.reshape(NW * n_chunks, CHUNK)
    col2 = col.reshape(NW * n_chunks, CHUNK)

    bw_pad = jnp.zeros((d_in, WP), jnp.float32).at[:, :r].set(B_w.T)
    a_pad = jnp.zeros((WP, d_out), jnp.float32).at[:r, :].set(A_w.T)
    zeros16 = jnp.zeros((npad, W), jnp.float32)
    zeros8 = jnp.zeros((npad, WP), jnp.float32)

    z = pl.pallas_call(
        _mm_body,
        out_shape=jax.ShapeDtypeStruct((n, WP), jnp.float32),
    )(x, bw_pad)

    degp = _make_deg_kernel(n_chunks, npad, rpt, trash)(row2, col2, zeros16)

    u = pl.pallas_call(
        _scale_body,
        out_shape=jax.ShapeDtypeStruct((npad, WP), jnp.float32),
    )(degp, z)

    tp = _make_prop_kernel(n_chunks, npad, rpt, trash)(row2, col2, zeros8, u)

    return pl.pallas_call(
        _final_body,
        out_shape=jax.ShapeDtypeStruct((n, d_out), jnp.float32),
    )(degp, tp, z, a_pad, bias.reshape(1, d_out))
